# baseline (speedup 1.0000x reference)
import functools

import numpy as np
import jax
import jax.numpy as jnp
from jax import lax
from jax.experimental import pallas as pl
from jax.experimental.pallas import tpu as pltpu
from jax.experimental.pallas import tpu_sc as plsc

_SEQ_LENGTH = 8192
_MASK_LENGTH = 2048
_D = 1024
_B = 4

_np_rng = np.random.RandomState(0)
_unmask_draw = _np_rng.randint(low=0, high=_SEQ_LENGTH, size=_MASK_LENGTH)
_UNMASK_BOOL = np.zeros(_SEQ_LENGTH, dtype=bool)
_UNMASK_BOOL[_unmask_draw] = True
_KEEP = np.where(_UNMASK_BOOL)[0].astype(np.int32)
_K = int(_KEEP.shape[0])

_info = plsc.get_sparse_core_info()
_NC = _info.num_cores
_NS = _info.num_subcores
_NW = _NC * _NS

_CHUNK_S = 15
_N_CHUNKS = 4
_S_PER_W = _CHUNK_S * _N_CHUNKS
_STRIDE = 57
_LAST_START = _K - _S_PER_W
_STARTS = np.minimum(np.arange(_NW) * _STRIDE, _LAST_START)
assert _STARTS[0] == 0 and _STARTS[-1] == _LAST_START
assert np.all(np.diff(_STARTS) <= _S_PER_W)

_IDX_TABLE = np.empty((_NW, _N_CHUNKS, 4 * _CHUNK_S), dtype=np.int32)
for _w in range(_NW):
    for _c in range(_N_CHUNKS):
        _s = _KEEP[_STARTS[_w] + _c * _CHUNK_S:
                   _STARTS[_w] + (_c + 1) * _CHUNK_S].astype(np.int64)
        _IDX_TABLE[_w, _c] = (
            np.arange(_B)[None, :] * _SEQ_LENGTH + _s[:, None]).reshape(-1)

_mesh = plsc.VectorSubcoreMesh(core_axis_name="c", subcore_axis_name="s")
_NBUF = 2


@functools.partial(
    pl.kernel,
    mesh=_mesh,
    out_type=jax.ShapeDtypeStruct((_K, _B, _D), jnp.float32),
    scratch_types=[
        pltpu.VMEM((_N_CHUNKS, 4 * _CHUNK_S), jnp.int32),
        pltpu.VMEM((_CHUNK_S, _B, _D), jnp.float32),
        pltpu.VMEM((_CHUNK_S, _B, _D), jnp.float32),
        pltpu.SemaphoreType.DMA,
        pltpu.SemaphoreType.DMA,
        pltpu.SemaphoreType.DMA,
        pltpu.SemaphoreType.DMA,
    ],
)
def _gather_rows(x_hbm, idx_hbm, out_hbm, idx_v,
                 buf0, buf1, gs0, gs1, os0, os1):
    wid = lax.axis_index("s") * _NC + lax.axis_index("c")
    base = lax.min(wid * _STRIDE, _LAST_START)
    pltpu.sync_copy(idx_hbm.at[wid], idx_v)
    bufs = (buf0, buf1)
    gsems = (gs0, gs1)
    osems = (os0, os1)
    look = _NBUF - 1
    gath = [None] * _N_CHUNKS
    last_scat = [None] * _NBUF
    for c in range(-look, _N_CHUNKS):
        f = c + look
        if 0 <= f < _N_CHUNKS:
            bb = f % _NBUF
            if last_scat[bb] is not None:
                last_scat[bb].wait()
                last_scat[bb] = None
            gath[f] = pltpu.async_copy(
                x_hbm.at[idx_v.at[f]],
                bufs[bb].reshape(4 * _CHUNK_S, _D), gsems[bb])
        if c >= 0:
            gath[c].wait()
            bb = c % _NBUF
            last_scat[bb] = pltpu.async_copy(
                bufs[bb], out_hbm.at[pl.ds(base + c * _CHUNK_S, _CHUNK_S)],
                osems[bb])
    for bb in range(_NBUF):
        if last_scat[bb] is not None:
            last_scat[bb].wait()


def kernel(x):
    x_flat = x.reshape(_B * _SEQ_LENGTH, _D)
    idx = jnp.asarray(_IDX_TABLE)
    out_sbd = _gather_rows(x_flat, idx)
    out = out_sbd.transpose(1, 0, 2)
    return (out, jnp.asarray(_UNMASK_BOOL))
